# Initial kernel scaffold; baseline (speedup 1.0000x reference)
import jax
import jax.numpy as jnp
from jax.experimental import pallas as pl


def kernel(x):
    raise NotImplementedError("write your pallas kernel here")


# speedup vs baseline: 3.2924x; 3.2924x over previous
import functools

import jax
import jax.numpy as jnp
from jax import lax
from jax.experimental import pallas as pl
from jax.experimental.pallas import tpu as pltpu
from jax.experimental.pallas import tpu_sc as plsc

_B = 1024
_N = 100000
_L = 10
_LANES = 16

_info = plsc.get_sparse_core_info()
_NC = _info.num_cores
_NW = _NC * _info.num_subcores
_ROWS_PER_W = _B // _NW

_mesh = plsc.VectorSubcoreMesh(core_axis_name="c", subcore_axis_name="s")


@functools.partial(
    pl.kernel,
    out_type=jax.ShapeDtypeStruct((_B, _N), jnp.float32),
    mesh=_mesh,
    scratch_types=[
        pltpu.VMEM((_ROWS_PER_W * _LANES,), jnp.int32),
        pltpu.VMEM((_N,), jnp.float32),
    ],
    compiler_params=pltpu.CompilerParams(needs_layout_passes=False),
)
def _ten_hot(x_hbm, out_hbm, xv, rowbuf):
    wid = lax.axis_index("s") * _NC + lax.axis_index("c")
    base = wid * _ROWS_PER_W

    pltpu.sync_copy(x_hbm.at[pl.ds(base * _LANES, _ROWS_PER_W * _LANES)], xv)

    zeros16 = jnp.zeros((_LANES,), jnp.float32)
    ones16 = jnp.ones((_LANES,), jnp.float32)

    def zero_body(i, c):
        rowbuf[pl.ds(i * _LANES, _LANES)] = zeros16
        return c

    lax.fori_loop(0, _N // _LANES, zero_body, 0)

    def row_body(r, c):
        toks = xv[pl.ds(r * _LANES, _LANES)]
        plsc.store_scatter(rowbuf, [toks], ones16)
        pltpu.sync_copy(rowbuf, out_hbm.at[base + r])
        plsc.store_scatter(rowbuf, [toks], zeros16)
        return c

    lax.fori_loop(0, _ROWS_PER_W, row_body, 0)


def kernel(x):
    xp = jnp.concatenate(
        [x, jnp.broadcast_to(x[:, :1], (_B, _LANES - _L))], axis=1
    )
    return _ten_hot(xp.reshape(-1))
